# Initial kernel scaffold; baseline (speedup 1.0000x reference)
import jax
import jax.numpy as jnp
from jax.experimental import pallas as pl


def kernel(w1, b1, w2, b2, w3, b3, w4, b4, fc_w, fc_b, x):
    raise NotImplementedError("write your pallas kernel here")


# speedup vs baseline: 2.0537x; 2.0537x over previous
import math

import jax
import jax.numpy as jnp
from jax.experimental import pallas as pl
from jax.experimental.pallas import tpu as pltpu

IMG_ROWS, IMG_COLS, NUM_CHAN = 8, 9, 4
C1, C2, C3, C4 = 64, 128, 256, 64
FC_OUT = 512
PH, PW = IMG_ROWS // 2, IMG_COLS // 2
FC_IN = C4 * PH * PW

B_TILE = 32
HF, WF = IMG_ROWS + 4, IMG_COLS + 4
MF = HF * WF
X = MF * B_TILE
K1 = 128
GUARD = (WF + 1) * B_TILE
EXT = 2 * GUARD + X


def _fused_cnn_kernel(x_ref, mask_ref, w1_ref, b1_ref, w2p_ref, w2s_ref,
                      b2_ref, w3p_ref, w3s_ref, b3_ref, w4_ref, b4_ref,
                      wfc_ref, bfc_ref, out_ref, xa_ref, xb_ref, feat_ref):
    G2, B = GUARD, B_TILE

    xa_ref[0:G2, :] = jnp.zeros((G2, 2 * C1), jnp.bfloat16)
    xa_ref[G2 + X - B:, :] = jnp.zeros((G2 + B, 2 * C1), jnp.bfloat16)
    xb_ref[0:G2, :] = jnp.zeros((G2, 2 * C2), jnp.bfloat16)
    xb_ref[G2 + X - B:, :] = jnp.zeros((G2 + B, 2 * C2), jnp.bfloat16)

    m64 = mask_ref[:, 0:C1]
    m128 = mask_ref[...]

    a1 = jnp.dot(x_ref[...], w1_ref[...], preferred_element_type=jnp.float32)
    a1 = (jnp.maximum(a1 + b1_ref[...], 0.0).astype(jnp.bfloat16)) * m64
    xa_ref[G2:G2 + X, 0:C1] = a1
    xa_ref[G2 - B:G2 - B + X, C1:2 * C1] = a1

    def conv3x3(src_ref, cin, cout, w_pair_ref, w_sngl_ref, b_ref):
        acc = jnp.zeros((X, cout), jnp.float32)
        for dh in range(3):
            rp = G2 + ((dh - 1) * WF - 1) * B
            rs = G2 + ((dh - 1) * WF + 1) * B
            acc = acc + jnp.dot(src_ref[rp:rp + X, 0:2 * cin], w_pair_ref[dh],
                                preferred_element_type=jnp.float32)
            acc = acc + jnp.dot(src_ref[rs:rs + X, 0:cin], w_sngl_ref[dh],
                                preferred_element_type=jnp.float32)
        return acc + b_ref[...]

    a2 = jnp.maximum(conv3x3(xa_ref, C1, C2, w2p_ref, w2s_ref, b2_ref), 0.0)
    a2 = a2.astype(jnp.bfloat16) * m128
    xb_ref[G2:G2 + X, 0:C2] = a2
    xb_ref[G2 - B:G2 - B + X, C2:2 * C2] = a2

    a3 = jnp.maximum(conv3x3(xb_ref, C2, C3, w3p_ref, w3s_ref, b3_ref), 0.0)
    a3 = a3.astype(jnp.bfloat16)

    a4 = jnp.maximum(jnp.dot(a3, w4_ref[...], preferred_element_type=jnp.float32)
                     + b4_ref[...], 0.0)
    xa_ref[G2:G2 + X, 0:C4] = a4.astype(jnp.bfloat16)

    for ph in range(PH):
        for pw in range(PW):
            p00 = (2 + 2 * ph) * WF + (2 + 2 * pw)
            r = G2 + p00 * B
            v = jnp.maximum(
                jnp.maximum(xa_ref[r:r + B, 0:C4],
                            xa_ref[r + B:r + 2 * B, 0:C4]),
                jnp.maximum(xa_ref[r + WF * B:r + WF * B + B, 0:C4],
                            xa_ref[r + (WF + 1) * B:r + (WF + 1) * B + B, 0:C4]))
            s = ph * PW + pw
            feat_ref[:, s * C4:(s + 1) * C4] = v
    out = jnp.dot(feat_ref[...], wfc_ref[...], preferred_element_type=jnp.float32)
    out_ref[...] = jnp.maximum(out + bfc_ref[...], 0.0)


def kernel(w1, b1, w2, b2, w3, b3, w4, b4, fc_w, fc_b, x):
    N = x.shape[0]
    G = -(-N // B_TILE)
    N_pad = G * B_TILE
    x = jnp.pad(x.astype(jnp.bfloat16),
                ((0, N_pad - N), (0, 0), (0, 0), (0, 0)))

    x_cl = jnp.transpose(x, (0, 2, 3, 1))
    x_fr = jnp.pad(x_cl, ((0, 0), (2, 2), (2, 2), (0, 0)))
    x_g = jnp.pad(x_fr, ((0, 0), (2, 2), (2, 2), (0, 0)))
    taps = [x_g[:, dh:dh + HF, dw:dw + WF, :]
            for dh in range(5) for dw in range(5)]
    x_im = jnp.concatenate(taps, axis=-1)
    x_im = jnp.pad(x_im, ((0, 0), (0, 0), (0, 0), (0, K1 - 25 * NUM_CHAN)))
    x_im = (x_im.reshape(G, B_TILE, MF, K1)
                .transpose(0, 2, 1, 3).reshape(G, X, K1))

    rr = jnp.arange(MF, dtype=jnp.int32) // WF
    cc = jnp.arange(MF, dtype=jnp.int32) % WF
    m = ((rr >= 2) & (rr < 2 + IMG_ROWS) &
         (cc >= 2) & (cc < 2 + IMG_COLS)).astype(jnp.bfloat16)
    mask = jnp.broadcast_to(m[:, None, None], (MF, B_TILE, C2)).reshape(X, C2)

    bf = jnp.bfloat16
    w1m = jnp.transpose(w1, (2, 3, 1, 0)).reshape(25 * NUM_CHAN, C1)
    w1m = jnp.pad(w1m, ((0, K1 - 25 * NUM_CHAN), (0, 0))).astype(bf)

    def pack3(w):
        t = jnp.transpose(w, (2, 3, 1, 0)).astype(bf)
        return jnp.concatenate([t[:, 0], t[:, 1]], axis=1), t[:, 2]

    w2p, w2s = pack3(w2)
    w3p, w3s = pack3(w3)
    w4m = w4[:, :, 0, 0].T.astype(bf)
    wfc = jnp.transpose(fc_w.reshape(FC_OUT, C4, PH, PW),
                        (2, 3, 1, 0)).reshape(FC_IN, FC_OUT).astype(bf)
    b1r, b2r, b3r, b4r = (v[None, :] for v in (b1, b2, b3, b4))
    bfc = fc_b[None, :]

    def full(a):
        return pl.BlockSpec(a.shape, lambda g, _nd=a.ndim: (0,) * _nd)

    flops = (G * 2 * X * (K1 * C1 + 9 * C1 * C2 + 9 * C2 * C3 + C3 * C4)
             + G * 2 * B_TILE * FC_IN * FC_OUT)
    bytes_accessed = 2 * (x_im.size + mask.size + w1m.size + w2p.size
                          + w2s.size + w3p.size + w3s.size + w4m.size
                          + wfc.size) + 4 * (b1r.size + b2r.size + b3r.size
                                             + b4r.size + bfc.size
                                             + N_pad * FC_OUT)

    out = pl.pallas_call(
        _fused_cnn_kernel,
        out_shape=jax.ShapeDtypeStruct((N_pad, FC_OUT), jnp.float32),
        grid=(G,),
        in_specs=[
            pl.BlockSpec((None, X, K1), lambda g: (g, 0, 0)),
            full(mask),
            full(w1m), full(b1r),
            full(w2p), full(w2s), full(b2r),
            full(w3p), full(w3s), full(b3r),
            full(w4m), full(b4r),
            full(wfc), full(bfc),
        ],
        out_specs=pl.BlockSpec((B_TILE, FC_OUT), lambda g: (g, 0)),
        scratch_shapes=[
            pltpu.VMEM((EXT, 2 * C1), jnp.bfloat16),
            pltpu.VMEM((EXT, 2 * C2), jnp.bfloat16),
            pltpu.VMEM((B_TILE, FC_IN), jnp.bfloat16),
        ],
        compiler_params=pltpu.CompilerParams(
            dimension_semantics=("parallel",),
            vmem_limit_bytes=48 * 1024 * 1024),
        cost_estimate=pl.CostEstimate(flops=flops, transcendentals=0,
                                      bytes_accessed=bytes_accessed),
    )(x_im, mask, w1m, b1r, w2p, w2s, b2r, w3p, w3s, b3r, w4m, b4r, wfc, bfc)
    return out[:N]


# speedup vs baseline: 6.5493x; 3.1891x over previous
import math

import jax
import jax.numpy as jnp
from jax.experimental import pallas as pl
from jax.experimental.pallas import tpu as pltpu

IMG_ROWS, IMG_COLS, NUM_CHAN = 8, 9, 4
C1, C2, C3, C4 = 64, 128, 256, 64
FC_OUT = 512
PH, PW = IMG_ROWS // 2, IMG_COLS // 2
FC_IN = C4 * PH * PW

B_TILE = 32
HF, WF = IMG_ROWS + 4, IMG_COLS + 4
MF = HF * WF
X = MF * B_TILE
K1 = 128
GUARD = (WF + 1) * B_TILE
EXT = 2 * GUARD + X


def _fused_cnn_kernel(x_ref, mask_ref, w1_ref, b1_ref, w2p_ref, w2s_ref,
                      b2_ref, w3p_ref, w3s_ref, b3_ref, w4_ref, b4_ref,
                      wfc_ref, bfc_ref, out_ref, xa_ref, xb_ref, feat_ref):
    G2, B = GUARD, B_TILE

    xa_ref[0:G2, :] = jnp.zeros((G2, 2 * C1), jnp.bfloat16)
    xa_ref[G2 + X - B:, :] = jnp.zeros((G2 + B, 2 * C1), jnp.bfloat16)
    xb_ref[0:G2, :] = jnp.zeros((G2, 2 * C2), jnp.bfloat16)
    xb_ref[G2 + X - B:, :] = jnp.zeros((G2 + B, 2 * C2), jnp.bfloat16)

    m64 = mask_ref[:, 0:C1]
    m128 = mask_ref[...]

    a1 = jnp.dot(x_ref[...], w1_ref[...], preferred_element_type=jnp.float32)
    a1 = (jnp.maximum(a1 + b1_ref[...], 0.0).astype(jnp.bfloat16)) * m64
    xa_ref[G2:G2 + X, 0:C1] = a1
    xa_ref[G2 - B:G2 - B + X, C1:2 * C1] = a1

    def conv3x3(src_ref, cin, cout, w_pair_ref, w_sngl_ref, b_ref):
        acc = jnp.zeros((X, cout), jnp.float32)
        for dh in range(3):
            rp = G2 + ((dh - 1) * WF - 1) * B
            rs = G2 + ((dh - 1) * WF + 1) * B
            acc = acc + jnp.dot(src_ref[rp:rp + X, 0:2 * cin], w_pair_ref[dh],
                                preferred_element_type=jnp.float32)
            acc = acc + jnp.dot(src_ref[rs:rs + X, 0:cin], w_sngl_ref[dh],
                                preferred_element_type=jnp.float32)
        return acc + b_ref[...]

    a2 = jnp.maximum(conv3x3(xa_ref, C1, C2, w2p_ref, w2s_ref, b2_ref), 0.0)
    a2 = a2.astype(jnp.bfloat16) * m128
    xb_ref[G2:G2 + X, 0:C2] = a2
    xb_ref[G2 - B:G2 - B + X, C2:2 * C2] = a2

    a3 = jnp.maximum(conv3x3(xb_ref, C2, C3, w3p_ref, w3s_ref, b3_ref), 0.0)
    a3 = a3.astype(jnp.bfloat16)

    a4 = jnp.maximum(jnp.dot(a3, w4_ref[...], preferred_element_type=jnp.float32)
                     + b4_ref[...], 0.0)
    xa_ref[G2:G2 + X, 0:C4] = a4.astype(jnp.bfloat16)

    for ph in range(PH):
        for pw in range(PW):
            p00 = (2 + 2 * ph) * WF + (2 + 2 * pw)
            r = G2 + p00 * B
            v = jnp.maximum(
                jnp.maximum(xa_ref[r:r + B, 0:C4],
                            xa_ref[r + B:r + 2 * B, 0:C4]),
                jnp.maximum(xa_ref[r + WF * B:r + WF * B + B, 0:C4],
                            xa_ref[r + (WF + 1) * B:r + (WF + 1) * B + B, 0:C4]))
            s = ph * PW + pw
            feat_ref[:, s * C4:(s + 1) * C4] = v
    out = jnp.dot(feat_ref[...], wfc_ref[...], preferred_element_type=jnp.float32)
    out_ref[...] = jnp.maximum(out + bfc_ref[...], 0.0)


def kernel(w1, b1, w2, b2, w3, b3, w4, b4, fc_w, fc_b, x):
    N = x.shape[0]
    G = -(-N // B_TILE)
    N_pad = G * B_TILE
    x = jnp.pad(x.astype(jnp.bfloat16),
                ((0, N_pad - N), (0, 0), (0, 0), (0, 0)))

    x_cl = jnp.transpose(x, (0, 2, 3, 1))
    x_fr = jnp.pad(x_cl, ((0, 0), (2, 2), (2, 2), (0, 0)))
    x_t = (x_fr.reshape(G, B_TILE, HF, WF, NUM_CHAN)
               .transpose(0, 2, 3, 1, 4))
    x_g = jnp.pad(x_t, ((0, 0), (2, 2), (2, 2), (0, 0), (0, 0)))
    taps = [x_g[:, dh:dh + HF, dw:dw + WF]
            for dh in range(5) for dw in range(5)]
    x_im = jnp.concatenate(taps, axis=-1)
    x_im = jnp.pad(x_im, ((0, 0), (0, 0), (0, 0), (0, 0),
                          (0, K1 - 25 * NUM_CHAN)))
    x_im = x_im.reshape(G, X, K1)

    rr = jnp.arange(MF, dtype=jnp.int32) // WF
    cc = jnp.arange(MF, dtype=jnp.int32) % WF
    m = ((rr >= 2) & (rr < 2 + IMG_ROWS) &
         (cc >= 2) & (cc < 2 + IMG_COLS)).astype(jnp.bfloat16)
    mask = jnp.broadcast_to(m[:, None, None], (MF, B_TILE, C2)).reshape(X, C2)

    bf = jnp.bfloat16
    w1m = jnp.transpose(w1, (2, 3, 1, 0)).reshape(25 * NUM_CHAN, C1)
    w1m = jnp.pad(w1m, ((0, K1 - 25 * NUM_CHAN), (0, 0))).astype(bf)

    def pack3(w):
        t = jnp.transpose(w, (2, 3, 1, 0)).astype(bf)
        return jnp.concatenate([t[:, 0], t[:, 1]], axis=1), t[:, 2]

    w2p, w2s = pack3(w2)
    w3p, w3s = pack3(w3)
    w4m = w4[:, :, 0, 0].T.astype(bf)
    wfc = jnp.transpose(fc_w.reshape(FC_OUT, C4, PH, PW),
                        (2, 3, 1, 0)).reshape(FC_IN, FC_OUT).astype(bf)
    b1r, b2r, b3r, b4r = (v[None, :] for v in (b1, b2, b3, b4))
    bfc = fc_b[None, :]

    def full(a):
        return pl.BlockSpec(a.shape, lambda g, _nd=a.ndim: (0,) * _nd)

    flops = (G * 2 * X * (K1 * C1 + 9 * C1 * C2 + 9 * C2 * C3 + C3 * C4)
             + G * 2 * B_TILE * FC_IN * FC_OUT)
    bytes_accessed = 2 * (x_im.size + mask.size + w1m.size + w2p.size
                          + w2s.size + w3p.size + w3s.size + w4m.size
                          + wfc.size) + 4 * (b1r.size + b2r.size + b3r.size
                                             + b4r.size + bfc.size
                                             + N_pad * FC_OUT)

    out = pl.pallas_call(
        _fused_cnn_kernel,
        out_shape=jax.ShapeDtypeStruct((N_pad, FC_OUT), jnp.float32),
        grid=(G,),
        in_specs=[
            pl.BlockSpec((None, X, K1), lambda g: (g, 0, 0)),
            full(mask),
            full(w1m), full(b1r),
            full(w2p), full(w2s), full(b2r),
            full(w3p), full(w3s), full(b3r),
            full(w4m), full(b4r),
            full(wfc), full(bfc),
        ],
        out_specs=pl.BlockSpec((B_TILE, FC_OUT), lambda g: (g, 0)),
        scratch_shapes=[
            pltpu.VMEM((EXT, 2 * C1), jnp.bfloat16),
            pltpu.VMEM((EXT, 2 * C2), jnp.bfloat16),
            pltpu.VMEM((B_TILE, FC_IN), jnp.bfloat16),
        ],
        compiler_params=pltpu.CompilerParams(
            dimension_semantics=("parallel",),
            vmem_limit_bytes=48 * 1024 * 1024),
        cost_estimate=pl.CostEstimate(flops=flops, transcendentals=0,
                                      bytes_accessed=bytes_accessed),
    )(x_im, mask, w1m, b1r, w2p, w2s, b2r, w3p, w3s, b3r, w4m, b4r, wfc, bfc)
    return out[:N]


# speedup vs baseline: 8.6731x; 1.3243x over previous
import jax
import jax.numpy as jnp
from jax.experimental import pallas as pl
from jax.experimental.pallas import tpu as pltpu

IMG_ROWS, IMG_COLS, NUM_CHAN = 8, 9, 4
C1, C2, C3, C4 = 64, 128, 256, 64
FC_OUT = 512
PH, PW = IMG_ROWS // 2, IMG_COLS // 2
FC_IN = C4 * PH * PW

B_TILE = 32
HF, WF = IMG_ROWS + 4, IMG_COLS + 4
MF = HF * WF
X = MF * B_TILE
KW1 = 32
G0 = 2 * WF * B_TILE
EXT0 = X + 2 * G0
GA = (WF + 1) * B_TILE
EXTA = X + 2 * GA


def _fused_cnn_kernel(x_ref, mask_ref, w1_ref, b1_ref, w2_ref, b2_ref,
                      w3_ref, b3_ref, w4_ref, b4_ref, wfc_ref, bfc_ref,
                      out_ref, xin_ref, xa_ref, xb_ref, feat_ref):
    B = B_TILE

    xin_ref[G0:G0 + G0, :] = jnp.zeros((G0, 5 * KW1), jnp.bfloat16)
    xin_ref[G0 + X - G0:G0 + X, :] = jnp.zeros((G0, 5 * KW1), jnp.bfloat16)
    xblk = x_ref[...]
    for k in range(5):
        off = (k - 2) * WF * B
        xin_ref[G0 - off:G0 - off + X, k * KW1:(k + 1) * KW1] = xblk
    a1 = jnp.dot(xin_ref[G0:G0 + X, :], w1_ref[...],
                 preferred_element_type=jnp.float32)
    a1 = jnp.maximum(a1 + b1_ref[...], 0.0).astype(jnp.bfloat16)
    a1 = a1 * mask_ref[:, 0:C1]

    def pack9(dst_ref, a, cin):
        dst_ref[GA:GA + GA, :] = jnp.zeros((GA, 9 * cin), jnp.bfloat16)
        dst_ref[GA + X - GA:GA + X, :] = jnp.zeros((GA, 9 * cin), jnp.bfloat16)
        for dh in range(3):
            for dw in range(3):
                j = dh * 3 + dw
                off = ((dh - 1) * WF + (dw - 1)) * B
                dst_ref[GA - off:GA - off + X, j * cin:(j + 1) * cin] = a

    pack9(xa_ref, a1, C1)
    a2 = jnp.dot(xa_ref[GA:GA + X, :], w2_ref[...],
                 preferred_element_type=jnp.float32)
    a2 = jnp.maximum(a2 + b2_ref[...], 0.0).astype(jnp.bfloat16)
    a2 = a2 * mask_ref[...]

    pack9(xb_ref, a2, C2)
    a3 = jnp.dot(xb_ref[GA:GA + X, :], w3_ref[...],
                 preferred_element_type=jnp.float32)
    a3 = jnp.maximum(a3 + b3_ref[...], 0.0).astype(jnp.bfloat16)

    a4 = jnp.maximum(jnp.dot(a3, w4_ref[...], preferred_element_type=jnp.float32)
                     + b4_ref[...], 0.0)
    xa_ref[GA:GA + X, 0:C4] = a4.astype(jnp.bfloat16)

    for ph in range(PH):
        for pw in range(PW):
            p00 = (2 + 2 * ph) * WF + (2 + 2 * pw)
            r = GA + p00 * B
            v = jnp.maximum(
                jnp.maximum(xa_ref[r:r + B, 0:C4],
                            xa_ref[r + B:r + 2 * B, 0:C4]),
                jnp.maximum(xa_ref[r + WF * B:r + WF * B + B, 0:C4],
                            xa_ref[r + (WF + 1) * B:r + (WF + 1) * B + B, 0:C4]))
            s = ph * PW + pw
            feat_ref[:, s * C4:(s + 1) * C4] = v
    out = jnp.dot(feat_ref[...], wfc_ref[...], preferred_element_type=jnp.float32)
    out_ref[...] = jnp.maximum(out + bfc_ref[...], 0.0)


def kernel(w1, b1, w2, b2, w3, b3, w4, b4, fc_w, fc_b, x):
    N = x.shape[0]
    G = -(-N // B_TILE)
    N_pad = G * B_TILE
    x = jnp.pad(x.astype(jnp.bfloat16),
                ((0, N_pad - N), (0, 0), (0, 0), (0, 0)))

    x_cl = jnp.transpose(x, (0, 2, 3, 1))
    x_fr = jnp.pad(x_cl, ((0, 0), (2, 2), (2, 2), (0, 0)))
    x_t = (x_fr.reshape(G, B_TILE, HF, WF, NUM_CHAN)
               .transpose(0, 2, 3, 1, 4))
    x_gw = jnp.pad(x_t, ((0, 0), (0, 0), (2, 2), (0, 0), (0, 0)))
    dw_taps = [x_gw[:, :, dw:dw + WF] for dw in range(5)]
    x_dw = jnp.concatenate(dw_taps, axis=-1)
    x_dw = jnp.pad(x_dw, ((0, 0),) * 4 + ((0, KW1 - 5 * NUM_CHAN),))
    x_dw = x_dw.reshape(G, X, KW1)

    rr = jnp.arange(MF, dtype=jnp.int32) // WF
    cc = jnp.arange(MF, dtype=jnp.int32) % WF
    m = ((rr >= 2) & (rr < 2 + IMG_ROWS) &
         (cc >= 2) & (cc < 2 + IMG_COLS)).astype(jnp.bfloat16)
    mask = jnp.broadcast_to(m[:, None, None], (MF, B_TILE, C2)).reshape(X, C2)

    bf = jnp.bfloat16
    w1t = jnp.transpose(w1, (2, 3, 1, 0)).reshape(5, 5 * NUM_CHAN, C1)
    w1m = jnp.pad(w1t, ((0, 0), (0, KW1 - 5 * NUM_CHAN), (0, 0)))
    w1m = w1m.reshape(5 * KW1, C1).astype(bf)

    def packw(w, cin, cout):
        return jnp.transpose(w, (2, 3, 1, 0)).reshape(9 * cin, cout).astype(bf)

    w2m = packw(w2, C1, C2)
    w3m = packw(w3, C2, C3)
    w4m = w4[:, :, 0, 0].T.astype(bf)
    wfc = jnp.transpose(fc_w.reshape(FC_OUT, C4, PH, PW),
                        (2, 3, 1, 0)).reshape(FC_IN, FC_OUT).astype(bf)
    b1r, b2r, b3r, b4r = (v[None, :] for v in (b1, b2, b3, b4))
    bfc = fc_b[None, :]

    def full(a):
        return pl.BlockSpec(a.shape, lambda g, _nd=a.ndim: (0,) * _nd)

    flops = (G * 2 * X * (5 * KW1 * C1 + 9 * C1 * C2 + 9 * C2 * C3 + C3 * C4)
             + G * 2 * B_TILE * FC_IN * FC_OUT)
    bytes_accessed = 2 * (x_dw.size + mask.size + w1m.size + w2m.size
                          + w3m.size + w4m.size + wfc.size) + 4 * N_pad * FC_OUT

    out = pl.pallas_call(
        _fused_cnn_kernel,
        out_shape=jax.ShapeDtypeStruct((N_pad, FC_OUT), jnp.float32),
        grid=(G,),
        in_specs=[
            pl.BlockSpec((None, X, KW1), lambda g: (g, 0, 0)),
            full(mask),
            full(w1m), full(b1r),
            full(w2m), full(b2r),
            full(w3m), full(b3r),
            full(w4m), full(b4r),
            full(wfc), full(bfc),
        ],
        out_specs=pl.BlockSpec((B_TILE, FC_OUT), lambda g: (g, 0)),
        scratch_shapes=[
            pltpu.VMEM((EXT0, 5 * KW1), jnp.bfloat16),
            pltpu.VMEM((EXTA, 9 * C1), jnp.bfloat16),
            pltpu.VMEM((EXTA, 9 * C2), jnp.bfloat16),
            pltpu.VMEM((B_TILE, FC_IN), jnp.bfloat16),
        ],
        compiler_params=pltpu.CompilerParams(
            dimension_semantics=("parallel",),
            vmem_limit_bytes=48 * 1024 * 1024),
        cost_estimate=pl.CostEstimate(flops=flops, transcendentals=0,
                                      bytes_accessed=bytes_accessed),
    )(x_dw, mask, w1m, b1r, w2m, b2r, w3m, b3r, w4m, b4r, wfc, bfc)
    return out[:N]
